# baseline (speedup 1.0000x reference)
import functools

import jax
import jax.numpy as jnp
from jax.experimental import pallas as pl
from jax.experimental.pallas import tpu as pltpu


def _dyn_slice(ref, start, blk):
    wsl = blk + 128
    fine = jax.lax.rem(start, 128)
    coarse = pl.multiple_of(start - fine, 128)
    win = ref[:, pl.ds(coarse, wsl)]
    rolled = pltpu.roll(win, jax.lax.rem(wsl - fine, wsl), axis=1)
    return rolled[:, :blk]


def _select_block(q, bank_blk, ext, blk, size, bs):
    col = jax.lax.broadcasted_iota(jnp.int32, (1, blk), 1)
    off = q + col
    wrapped = off >= size
    offm = jnp.where(wrapped, off - size, off)
    mask = offm < bs
    start1 = blk + jnp.minimum(q, bs)
    start2 = jnp.clip(q - (size - blk), 0, blk)
    part1 = _dyn_slice(ext, start1, blk)
    part2 = _dyn_slice(ext, start2, blk)
    g = jnp.where(wrapped, part2, part1)
    return jnp.where(mask, g, bank_blk)


def _body(ptr_ref, bank_ref, ext_ref, bl_ref, lext_ref,
          pre_ref, new_ref, lpre_ref, lnew_ref,
          ext_v, lext_v, in_v, out_v, lin_v, lout_v, sems, hsems,
          *, blk, size, bs, nhit):
    p = ptr_ref[0]
    nb = size // blk

    cp_pre = pltpu.make_async_copy(bank_ref, pre_ref, sems.at[0])
    cp_new = pltpu.make_async_copy(bank_ref, new_ref, sems.at[1])
    cp_lpre = pltpu.make_async_copy(bl_ref, lpre_ref, sems.at[2])
    cp_lnew = pltpu.make_async_copy(bl_ref, lnew_ref, sems.at[3])
    cp_pre.start()
    cp_new.start()
    cp_lpre.start()
    cp_lnew.start()

    cp_ext = pltpu.make_async_copy(ext_ref, ext_v, sems.at[4])
    cp_lext = pltpu.make_async_copy(lext_ref, lext_v, sems.at[5])
    cp_ext.start()
    cp_lext.start()

    k0 = jax.lax.div(p, blk)
    loads = []
    for h in range(nhit):
        k = jax.lax.rem(k0 + h, nb)
        s = pl.multiple_of(k * blk, blk)
        ld = pltpu.make_async_copy(bank_ref.at[:, pl.ds(s, blk)],
                                   in_v.at[h], hsems.at[h])
        ld.start()
        lld = pltpu.make_async_copy(bl_ref.at[:, pl.ds(s, blk)],
                                    lin_v.at[h], hsems.at[nhit + h])
        lld.start()
        loads.append((k, s, ld, lld))

    cp_ext.wait()
    cp_lext.wait()
    for h, (k, s, ld, lld) in enumerate(loads):
        ld.wait()
        lld.wait()
        q = jax.lax.rem(k * blk - p + size, size)
        out_v[h] = _select_block(q, in_v[h], ext_v, blk, size, bs)
        lout_v[h] = _select_block(q, lin_v[h], lext_v, blk, size, bs)

    cp_new.wait()
    cp_lnew.wait()
    stores = []
    for h, (k, s, ld, lld) in enumerate(loads):
        st = pltpu.make_async_copy(out_v.at[h], new_ref.at[:, pl.ds(s, blk)],
                                   hsems.at[h])
        st.start()
        lst = pltpu.make_async_copy(lout_v.at[h],
                                    lnew_ref.at[:, pl.ds(s, blk)],
                                    hsems.at[nhit + h])
        lst.start()
        stores.append((st, lst))
    for st, lst in stores:
        st.wait()
        lst.wait()
    cp_pre.wait()
    cp_lpre.wait()


def kernel(output, labels, bank, bank_labels, ptr):
    dim, size = bank.shape
    bs = output.shape[0]
    blk = 4096
    nhit = (bs - 1) // blk + 2
    w = bs + 2 * blk + 128

    out_t = output.T.astype(jnp.float32)
    zpad = jnp.zeros((dim, blk), jnp.float32)
    zpad_r = jnp.zeros((dim, blk + 128), jnp.float32)
    ext = jnp.concatenate([zpad, out_t, zpad_r], axis=1)
    lzpad = jnp.zeros((1, blk), jnp.int32)
    lzpad_r = jnp.zeros((1, blk + 128), jnp.int32)
    lext = jnp.concatenate([lzpad, labels[None, :], lzpad_r], axis=1)
    bl2 = bank_labels.reshape(1, size)
    ptr_arr = jnp.asarray(ptr, jnp.int32).reshape(1)

    out_shapes = (
        jax.ShapeDtypeStruct((dim, size), jnp.float32),
        jax.ShapeDtypeStruct((dim, size), jnp.float32),
        jax.ShapeDtypeStruct((1, size), jnp.int32),
        jax.ShapeDtypeStruct((1, size), jnp.int32),
    )
    any_spec = pl.BlockSpec(memory_space=pl.ANY)
    fn = pl.pallas_call(
        functools.partial(_body, blk=blk, size=size, bs=bs, nhit=nhit),
        in_specs=[
            pl.BlockSpec(memory_space=pltpu.SMEM),
            any_spec, any_spec, any_spec, any_spec,
        ],
        out_specs=[any_spec, any_spec, any_spec, any_spec],
        out_shape=out_shapes,
        scratch_shapes=[
            pltpu.VMEM((dim, w), jnp.float32),
            pltpu.VMEM((1, w), jnp.int32),
            pltpu.VMEM((nhit, dim, blk), jnp.float32),
            pltpu.VMEM((nhit, dim, blk), jnp.float32),
            pltpu.VMEM((nhit, 1, blk), jnp.int32),
            pltpu.VMEM((nhit, 1, blk), jnp.int32),
            pltpu.SemaphoreType.DMA((8,)),
            pltpu.SemaphoreType.DMA((2 * nhit,)),
        ],
    )
    pre, new, lpre2, lnew2 = fn(ptr_arr, bank, ext, bl2, lext)
    return (output, pre, lpre2.reshape(size), new, lnew2.reshape(size))


# speedup vs baseline: 1.0009x; 1.0009x over previous
import functools

import jax
import jax.numpy as jnp
from jax.experimental import pallas as pl
from jax.experimental.pallas import tpu as pltpu


def _dyn_slice(ref, start, blk):
    wsl = blk + 128
    fine = jax.lax.rem(start, 128)
    coarse = pl.multiple_of(start - fine, 128)
    win = ref[:, pl.ds(coarse, wsl)]
    rolled = pltpu.roll(win, jax.lax.rem(wsl - fine, wsl), axis=1)
    return rolled[:, :blk]


def _select_block(q, bank_blk, ext, blk, size, bs):
    col = jax.lax.broadcasted_iota(jnp.int32, (1, blk), 1)
    off = q + col
    wrapped = off >= size
    offm = jnp.where(wrapped, off - size, off)
    mask = offm < bs
    start1 = blk + jnp.minimum(q, bs)
    start2 = jnp.clip(q - (size - blk), 0, blk)
    part1 = _dyn_slice(ext, start1, blk)
    part2 = _dyn_slice(ext, start2, blk)
    g = jnp.where(wrapped, part2, part1)
    return jnp.where(mask, g, bank_blk)


def _body(ptr_ref, bank_ref, ext_ref, bl_ref, lext_ref,
          pre_ref, new_ref, lpre_ref, lnew_ref,
          ext_v, lext_v, in_v, out_v, lin_v, lout_v, sems, hsems,
          *, blk, size, bs, nhit):
    p = ptr_ref[0]
    nb = size // blk

    dense = []
    for c in range(nb):
        cs = c * blk
        cp1 = pltpu.make_async_copy(bank_ref.at[:, pl.ds(cs, blk)],
                                    pre_ref.at[:, pl.ds(cs, blk)],
                                    sems.at[6])
        cp2 = pltpu.make_async_copy(bank_ref.at[:, pl.ds(cs, blk)],
                                    new_ref.at[:, pl.ds(cs, blk)],
                                    sems.at[7])
        cp1.start()
        cp2.start()
        dense.append((cp1, cp2))
    cp_lpre = pltpu.make_async_copy(bl_ref, lpre_ref, sems.at[2])
    cp_lnew = pltpu.make_async_copy(bl_ref, lnew_ref, sems.at[3])
    cp_lpre.start()
    cp_lnew.start()

    cp_ext = pltpu.make_async_copy(ext_ref, ext_v, sems.at[4])
    cp_lext = pltpu.make_async_copy(lext_ref, lext_v, sems.at[5])
    cp_ext.start()
    cp_lext.start()

    k0 = jax.lax.div(p, blk)
    loads = []
    for h in range(nhit):
        k = jax.lax.rem(k0 + h, nb)
        s = pl.multiple_of(k * blk, blk)
        ld = pltpu.make_async_copy(bank_ref.at[:, pl.ds(s, blk)],
                                   in_v.at[h], hsems.at[h])
        ld.start()
        lld = pltpu.make_async_copy(bl_ref.at[:, pl.ds(s, blk)],
                                    lin_v.at[h], hsems.at[nhit + h])
        lld.start()
        loads.append((k, s, ld, lld))

    cp_ext.wait()
    cp_lext.wait()
    for h, (k, s, ld, lld) in enumerate(loads):
        ld.wait()
        lld.wait()
        q = jax.lax.rem(k * blk - p + size, size)
        out_v[h] = _select_block(q, in_v[h], ext_v, blk, size, bs)
        lout_v[h] = _select_block(q, lin_v[h], lext_v, blk, size, bs)

    for cp1, cp2 in dense:
        cp1.wait()
        cp2.wait()
    cp_lnew.wait()
    stores = []
    for h, (k, s, ld, lld) in enumerate(loads):
        st = pltpu.make_async_copy(out_v.at[h], new_ref.at[:, pl.ds(s, blk)],
                                   hsems.at[h])
        st.start()
        lst = pltpu.make_async_copy(lout_v.at[h],
                                    lnew_ref.at[:, pl.ds(s, blk)],
                                    hsems.at[nhit + h])
        lst.start()
        stores.append((st, lst))
    for st, lst in stores:
        st.wait()
        lst.wait()
    cp_lpre.wait()


def kernel(output, labels, bank, bank_labels, ptr):
    dim, size = bank.shape
    bs = output.shape[0]
    blk = 4096
    nhit = (bs - 1) // blk + 2
    w = bs + 2 * blk + 128

    out_t = output.T.astype(jnp.float32)
    zpad = jnp.zeros((dim, blk), jnp.float32)
    zpad_r = jnp.zeros((dim, blk + 128), jnp.float32)
    ext = jnp.concatenate([zpad, out_t, zpad_r], axis=1)
    lzpad = jnp.zeros((1, blk), jnp.int32)
    lzpad_r = jnp.zeros((1, blk + 128), jnp.int32)
    lext = jnp.concatenate([lzpad, labels[None, :], lzpad_r], axis=1)
    bl2 = bank_labels.reshape(1, size)
    ptr_arr = jnp.asarray(ptr, jnp.int32).reshape(1)

    out_shapes = (
        jax.ShapeDtypeStruct((dim, size), jnp.float32),
        jax.ShapeDtypeStruct((dim, size), jnp.float32),
        jax.ShapeDtypeStruct((1, size), jnp.int32),
        jax.ShapeDtypeStruct((1, size), jnp.int32),
    )
    any_spec = pl.BlockSpec(memory_space=pl.ANY)
    fn = pl.pallas_call(
        functools.partial(_body, blk=blk, size=size, bs=bs, nhit=nhit),
        in_specs=[
            pl.BlockSpec(memory_space=pltpu.SMEM),
            any_spec, any_spec, any_spec, any_spec,
        ],
        out_specs=[any_spec, any_spec, any_spec, any_spec],
        out_shape=out_shapes,
        scratch_shapes=[
            pltpu.VMEM((dim, w), jnp.float32),
            pltpu.VMEM((1, w), jnp.int32),
            pltpu.VMEM((nhit, dim, blk), jnp.float32),
            pltpu.VMEM((nhit, dim, blk), jnp.float32),
            pltpu.VMEM((nhit, 1, blk), jnp.int32),
            pltpu.VMEM((nhit, 1, blk), jnp.int32),
            pltpu.SemaphoreType.DMA((8,)),
            pltpu.SemaphoreType.DMA((2 * nhit,)),
        ],
    )
    pre, new, lpre2, lnew2 = fn(ptr_arr, bank, ext, bl2, lext)
    return (output, pre, lpre2.reshape(size), new, lnew2.reshape(size))


# speedup vs baseline: 55.0548x; 55.0041x over previous
import functools

import jax
import jax.numpy as jnp
from jax.experimental import pallas as pl
from jax.experimental.pallas import tpu as pltpu


def _dyn_slice(ref, start, blk):
    wsl = blk + 128
    fine = jax.lax.rem(start, 128)
    coarse = pl.multiple_of(start - fine, 128)
    win = ref[:, pl.ds(coarse, wsl)]
    rolled = pltpu.roll(win, jax.lax.rem(wsl - fine, wsl), axis=1)
    return rolled[:, :blk]


def _select_block(q, bank_blk, ext, blk, size, bs):
    col = jax.lax.broadcasted_iota(jnp.int32, (1, blk), 1)
    off = q + col
    wrapped = off >= size
    offm = jnp.where(wrapped, off - size, off)
    mask = offm < bs
    start1 = blk + jnp.minimum(q, bs)
    start2 = jnp.clip(q - (size - blk), 0, blk)
    part1 = _dyn_slice(ext, start1, blk)
    part2 = _dyn_slice(ext, start2, blk)
    g = jnp.where(wrapped, part2, part1)
    return jnp.where(mask, g, bank_blk)


def _body(ptr_ref, bank_ref, ext_ref, bl_ref, lext_ref,
          pre_ref, new_ref, lpre_ref, lnew_ref,
          ext_v, lext_v, in_v, out_v, lin_v, lout_v, sems, hsems,
          dense_v, dsem, dsem_pre, dsem_new, lbl_v,
          *, blk, size, bs, nhit, nbuf):
    p = ptr_ref[0]
    nb = size // blk

    cp_lbl = pltpu.make_async_copy(bl_ref, lbl_v, sems.at[0])
    cp_lbl.start()
    cp_ext = pltpu.make_async_copy(ext_ref, ext_v, sems.at[4])
    cp_lext = pltpu.make_async_copy(lext_ref, lext_v, sems.at[5])
    cp_ext.start()
    cp_lext.start()

    k0 = jax.lax.div(p, blk)
    hits = []
    for h in range(nhit):
        k = jax.lax.rem(k0 + h, nb)
        s = pl.multiple_of(k * blk, blk)
        ld = pltpu.make_async_copy(bank_ref.at[:, pl.ds(s, blk)],
                                   in_v.at[h], hsems.at[h])
        ld.start()
        lld = pltpu.make_async_copy(bl_ref.at[:, pl.ds(s, blk)],
                                    lin_v.at[h], hsems.at[nhit + h])
        lld.start()
        hits.append((k, s, ld, lld))

    cp_lbl.wait()
    cp_lpre = pltpu.make_async_copy(lbl_v, lpre_ref, sems.at[2])
    cp_lnew = pltpu.make_async_copy(lbl_v, lnew_ref, sems.at[3])
    cp_lpre.start()
    cp_lnew.start()

    def mk_load(c):
        return pltpu.make_async_copy(bank_ref.at[:, pl.ds(c * blk, blk)],
                                     dense_v.at[c % nbuf], dsem.at[c % nbuf])

    def mk_store(c, dst_ref, sem):
        return pltpu.make_async_copy(dense_v.at[c % nbuf],
                                     dst_ref.at[:, pl.ds(c * blk, blk)],
                                     sem.at[c % nbuf])

    lag = nbuf - 2
    for c in range(nb):
        if c >= nbuf:
            mk_store(c - nbuf, pre_ref, dsem_pre).wait()
            mk_store(c - nbuf, new_ref, dsem_new).wait()
        mk_load(c).start()
        if c >= lag:
            d = c - lag
            mk_load(d).wait()
            mk_store(d, pre_ref, dsem_pre).start()
            mk_store(d, new_ref, dsem_new).start()
    for d in range(nb - lag, nb):
        mk_load(d).wait()
        mk_store(d, pre_ref, dsem_pre).start()
        mk_store(d, new_ref, dsem_new).start()
    dense_tail = min(nbuf, nb)

    cp_ext.wait()
    cp_lext.wait()
    for h, (k, s, ld, lld) in enumerate(hits):
        ld.wait()
        lld.wait()
        q = jax.lax.rem(k * blk - p + size, size)
        out_v[h] = _select_block(q, in_v[h], ext_v, blk, size, bs)
        lout_v[h] = _select_block(q, lin_v[h], lext_v, blk, size, bs)

    for d in range(nb - dense_tail, nb):
        mk_store(d, pre_ref, dsem_pre).wait()
        mk_store(d, new_ref, dsem_new).wait()
    cp_lnew.wait()
    stores = []
    for h, (k, s, ld, lld) in enumerate(hits):
        st = pltpu.make_async_copy(out_v.at[h], new_ref.at[:, pl.ds(s, blk)],
                                   hsems.at[h])
        st.start()
        lst = pltpu.make_async_copy(lout_v.at[h],
                                    lnew_ref.at[:, pl.ds(s, blk)],
                                    hsems.at[nhit + h])
        lst.start()
        stores.append((st, lst))
    for st, lst in stores:
        st.wait()
        lst.wait()
    cp_lpre.wait()


def kernel(output, labels, bank, bank_labels, ptr):
    dim, size = bank.shape
    bs = output.shape[0]
    blk = 4096
    nhit = (bs - 1) // blk + 2
    w = bs + 2 * blk + 128

    out_t = output.T.astype(jnp.float32)
    zpad = jnp.zeros((dim, blk), jnp.float32)
    zpad_r = jnp.zeros((dim, blk + 128), jnp.float32)
    ext = jnp.concatenate([zpad, out_t, zpad_r], axis=1)
    lzpad = jnp.zeros((1, blk), jnp.int32)
    lzpad_r = jnp.zeros((1, blk + 128), jnp.int32)
    lext = jnp.concatenate([lzpad, labels[None, :], lzpad_r], axis=1)
    bl2 = bank_labels.reshape(1, size)
    ptr_arr = jnp.asarray(ptr, jnp.int32).reshape(1)

    out_shapes = (
        jax.ShapeDtypeStruct((dim, size), jnp.float32),
        jax.ShapeDtypeStruct((dim, size), jnp.float32),
        jax.ShapeDtypeStruct((1, size), jnp.int32),
        jax.ShapeDtypeStruct((1, size), jnp.int32),
    )
    any_spec = pl.BlockSpec(memory_space=pl.ANY)
    nbuf = 4
    fn = pl.pallas_call(
        functools.partial(_body, blk=blk, size=size, bs=bs, nhit=nhit,
                          nbuf=nbuf),
        in_specs=[
            pl.BlockSpec(memory_space=pltpu.SMEM),
            any_spec, any_spec, any_spec, any_spec,
        ],
        out_specs=[any_spec, any_spec, any_spec, any_spec],
        out_shape=out_shapes,
        scratch_shapes=[
            pltpu.VMEM((dim, w), jnp.float32),
            pltpu.VMEM((1, w), jnp.int32),
            pltpu.VMEM((nhit, dim, blk), jnp.float32),
            pltpu.VMEM((nhit, dim, blk), jnp.float32),
            pltpu.VMEM((nhit, 1, blk), jnp.int32),
            pltpu.VMEM((nhit, 1, blk), jnp.int32),
            pltpu.SemaphoreType.DMA((8,)),
            pltpu.SemaphoreType.DMA((2 * nhit,)),
            pltpu.VMEM((nbuf, dim, blk), jnp.float32),
            pltpu.SemaphoreType.DMA((nbuf,)),
            pltpu.SemaphoreType.DMA((nbuf,)),
            pltpu.SemaphoreType.DMA((nbuf,)),
            pltpu.VMEM((1, size), jnp.int32),
        ],
    )
    pre, new, lpre2, lnew2 = fn(ptr_arr, bank, ext, bl2, lext)
    return (output, pre, lpre2.reshape(size), new, lnew2.reshape(size))


# speedup vs baseline: 56.8111x; 1.0319x over previous
import functools

import jax
import jax.numpy as jnp
from jax.experimental import pallas as pl
from jax.experimental.pallas import tpu as pltpu


def _dyn_slice(ref, start, blk):
    wsl = blk + 128
    fine = jax.lax.rem(start, 128)
    coarse = pl.multiple_of(start - fine, 128)
    win = ref[:, pl.ds(coarse, wsl)]
    rolled = pltpu.roll(win, jax.lax.rem(wsl - fine, wsl), axis=1)
    return rolled[:, :blk]


def _select_block(q, bank_blk, ext, blk, size, bs):
    col = jax.lax.broadcasted_iota(jnp.int32, (1, blk), 1)
    off = q + col
    wrapped = off >= size
    offm = jnp.where(wrapped, off - size, off)
    mask = offm < bs
    start1 = blk + jnp.minimum(q, bs)
    start2 = jnp.clip(q - (size - blk), 0, blk)
    part1 = _dyn_slice(ext, start1, blk)
    part2 = _dyn_slice(ext, start2, blk)
    g = jnp.where(wrapped, part2, part1)
    return jnp.where(mask, g, bank_blk)


def _body(ptr_ref, bank_ref, ext_ref, bl_ref, lext_ref,
          pre_ref, new_ref, lpre_ref, lnew_ref,
          ext_v, lext_v, in_v, out_v, lin_v, lout_v, sems, hsems,
          dense_v, dsem, dsem_pre, dsem_new, lbl_v,
          *, blk, size, bs, nhit, nbuf):
    p = ptr_ref[0]
    nb = size // blk

    cp_lbl = pltpu.make_async_copy(bl_ref, lbl_v, sems.at[0])
    cp_lbl.start()
    cp_ext = pltpu.make_async_copy(ext_ref, ext_v, sems.at[4])
    cp_lext = pltpu.make_async_copy(lext_ref, lext_v, sems.at[5])
    cp_ext.start()
    cp_lext.start()

    k0 = jax.lax.div(p, blk)
    hits = []
    for h in range(nhit):
        k = jax.lax.rem(k0 + h, nb)
        s = pl.multiple_of(k * blk, blk)
        ld = pltpu.make_async_copy(bank_ref.at[:, pl.ds(s, blk)],
                                   in_v.at[h], hsems.at[h])
        ld.start()
        lld = pltpu.make_async_copy(bl_ref.at[:, pl.ds(s, blk)],
                                    lin_v.at[h], hsems.at[nhit + h])
        lld.start()
        hits.append((k, s, ld, lld))

    cp_lbl.wait()
    cp_lpre = pltpu.make_async_copy(lbl_v, lpre_ref, sems.at[2])
    cp_lnew = pltpu.make_async_copy(lbl_v, lnew_ref, sems.at[3])
    cp_lpre.start()
    cp_lnew.start()

    def mk_load(c):
        return pltpu.make_async_copy(bank_ref.at[:, pl.ds(c * blk, blk)],
                                     dense_v.at[c % nbuf], dsem.at[c % nbuf])

    def mk_store(c, dst_ref, sem):
        return pltpu.make_async_copy(dense_v.at[c % nbuf],
                                     dst_ref.at[:, pl.ds(c * blk, blk)],
                                     sem.at[c % nbuf])

    lag = nbuf - 2
    for c in range(nb):
        if c >= nbuf:
            mk_store(c - nbuf, pre_ref, dsem_pre).wait()
            mk_store(c - nbuf, new_ref, dsem_new).wait()
        mk_load(c).start()
        if c >= lag:
            d = c - lag
            mk_load(d).wait()
            mk_store(d, pre_ref, dsem_pre).start()
            mk_store(d, new_ref, dsem_new).start()
    for d in range(nb - lag, nb):
        mk_load(d).wait()
        mk_store(d, pre_ref, dsem_pre).start()
        mk_store(d, new_ref, dsem_new).start()
    dense_tail = min(nbuf, nb)

    cp_ext.wait()
    cp_lext.wait()
    for h, (k, s, ld, lld) in enumerate(hits):
        ld.wait()
        lld.wait()
        q = jax.lax.rem(k * blk - p + size, size)
        out_v[h] = _select_block(q, in_v[h], ext_v, blk, size, bs)
        lout_v[h] = _select_block(q, lin_v[h], lext_v, blk, size, bs)

    for d in range(nb - dense_tail, nb):
        mk_store(d, pre_ref, dsem_pre).wait()
        mk_store(d, new_ref, dsem_new).wait()
    cp_lnew.wait()
    stores = []
    for h, (k, s, ld, lld) in enumerate(hits):
        st = pltpu.make_async_copy(out_v.at[h], new_ref.at[:, pl.ds(s, blk)],
                                   hsems.at[h])
        st.start()
        lst = pltpu.make_async_copy(lout_v.at[h],
                                    lnew_ref.at[:, pl.ds(s, blk)],
                                    hsems.at[nhit + h])
        lst.start()
        stores.append((st, lst))
    for st, lst in stores:
        st.wait()
        lst.wait()
    cp_lpre.wait()


def kernel(output, labels, bank, bank_labels, ptr):
    dim, size = bank.shape
    bs = output.shape[0]
    blk = 4096
    nhit = (bs - 1) // blk + 2
    w = bs + 2 * blk + 128

    out_t = output.T.astype(jnp.float32)
    zpad = jnp.zeros((dim, blk), jnp.float32)
    zpad_r = jnp.zeros((dim, blk + 128), jnp.float32)
    ext = jnp.concatenate([zpad, out_t, zpad_r], axis=1)
    lzpad = jnp.zeros((1, blk), jnp.int32)
    lzpad_r = jnp.zeros((1, blk + 128), jnp.int32)
    lext = jnp.concatenate([lzpad, labels[None, :], lzpad_r], axis=1)
    bl2 = bank_labels.reshape(1, size)
    ptr_arr = jnp.asarray(ptr, jnp.int32).reshape(1)

    out_shapes = (
        jax.ShapeDtypeStruct((dim, size), jnp.float32),
        jax.ShapeDtypeStruct((dim, size), jnp.float32),
        jax.ShapeDtypeStruct((1, size), jnp.int32),
        jax.ShapeDtypeStruct((1, size), jnp.int32),
    )
    any_spec = pl.BlockSpec(memory_space=pl.ANY)
    nbuf = 8
    fn = pl.pallas_call(
        functools.partial(_body, blk=blk, size=size, bs=bs, nhit=nhit,
                          nbuf=nbuf),
        in_specs=[
            pl.BlockSpec(memory_space=pltpu.SMEM),
            any_spec, any_spec, any_spec, any_spec,
        ],
        out_specs=[any_spec, any_spec, any_spec, any_spec],
        out_shape=out_shapes,
        scratch_shapes=[
            pltpu.VMEM((dim, w), jnp.float32),
            pltpu.VMEM((1, w), jnp.int32),
            pltpu.VMEM((nhit, dim, blk), jnp.float32),
            pltpu.VMEM((nhit, dim, blk), jnp.float32),
            pltpu.VMEM((nhit, 1, blk), jnp.int32),
            pltpu.VMEM((nhit, 1, blk), jnp.int32),
            pltpu.SemaphoreType.DMA((8,)),
            pltpu.SemaphoreType.DMA((2 * nhit,)),
            pltpu.VMEM((nbuf, dim, blk), jnp.float32),
            pltpu.SemaphoreType.DMA((nbuf,)),
            pltpu.SemaphoreType.DMA((nbuf,)),
            pltpu.SemaphoreType.DMA((nbuf,)),
            pltpu.VMEM((1, size), jnp.int32),
        ],
    )
    pre, new, lpre2, lnew2 = fn(ptr_arr, bank, ext, bl2, lext)
    return (output, pre, lpre2.reshape(size), new, lnew2.reshape(size))
